# Initial kernel scaffold; baseline (speedup 1.0000x reference)
import jax
import jax.numpy as jnp
from jax.experimental import pallas as pl


def kernel(x, edge_index, W1e, b1e, W2e, b2e, W1d, b1d, W2d, b2d):
    raise NotImplementedError("write your pallas kernel here")


# speedup vs baseline: 5.8183x; 5.8183x over previous
import functools

import jax
import jax.numpy as jnp
from jax import lax
from jax.experimental import pallas as pl
from jax.experimental.pallas import tpu as pltpu
from jax.experimental.pallas import tpu_sc as plsc

N = 10000
E = 320000
NP = 10240
NC = 2
NS = 16
CELLS = NP * NP
ROW_W = 80
EROWS = E // ROW_W
TROWS = EROWS // NS
ZB = 20480
NPT = NP // NS
HALF = NP // 2

_mesh = plsc.VectorSubcoreMesh(core_axis_name="c", subcore_axis_name="s")
_sc_params = pltpu.CompilerParams(use_tc_tiling_on_sc=False,
                                  needs_layout_passes=False)


def _rsqrt_vec(x):
    bits = lax.bitcast_convert_type(x, jnp.int32)
    y = lax.bitcast_convert_type(
        jnp.int32(0x5F3759DF) - lax.shift_right_logical(bits, 1), jnp.float32)
    for _ in range(3):
        y = y * (1.5 - 0.5 * x * y * y)
    return y


@functools.partial(
    pl.kernel,
    out_type=(jax.ShapeDtypeStruct((NP,), jnp.float32),
              jax.ShapeDtypeStruct((NP,), jnp.float32)),
    mesh=_mesh,
    scratch_types=[
        pltpu.VMEM((TROWS, ROW_W), jnp.int32),
        pltpu.VMEM((ROW_W,), jnp.float32),
        pltpu.VMEM((NPT,), jnp.float32),
        pltpu.VMEM((NPT,), jnp.float32),
        pltpu.VMEM((NPT,), jnp.float32),
        pltpu.VMEM_SHARED((NP,), jnp.float32),
        pltpu.SemaphoreType.DMA,
    ],
    compiler_params=_sc_params,
)
def _hist_kernel(dst_hbm, dinv_hbm, d2_hbm, dstb, ones_v, hv, dv, d2v, hist,
                 sem):
    c = lax.axis_index("c")
    s = lax.axis_index("s")

    @pl.when(c == 0)
    def _():
        nrows = TROWS
        pltpu.sync_copy(dst_hbm.at[s], dstb)
        for l in range(ROW_W // 16):
            ones_v[pl.ds(l * 16, 16)] = jnp.full((16,), 1.0, jnp.float32)

        def zslice(i, carry):
            hv[pl.ds(i * 16, 16)] = jnp.zeros((16,), jnp.float32)
            return carry
        lax.fori_loop(0, NPT // 16, zslice, 0)
        pltpu.sync_copy(hv, hist.at[pl.ds(s * NPT, NPT)])
        plsc.subcore_barrier()

        def fire(j, carry):
            pltpu.async_copy(ones_v, hist.at[dstb.at[j]], sem, add=True)
            return carry
        lax.fori_loop(0, nrows, fire, 0)

        def drain(j, carry):
            pltpu.make_async_copy(ones_v, hist.at[dstb.at[0]], sem).wait()
            return carry
        lax.fori_loop(0, nrows, drain, 0)
        plsc.subcore_barrier()

        pltpu.sync_copy(hist.at[pl.ds(s * NPT, NPT)], hv)

        def body(i, carry):
            deg = hv[pl.ds(i * 16, 16)] + 1.0
            r = _rsqrt_vec(deg)
            dv[pl.ds(i * 16, 16)] = r
            d2v[pl.ds(i * 16, 16)] = r * r
            return carry
        lax.fori_loop(0, NPT // 16, body, 0)
        pltpu.sync_copy(dv, dinv_hbm.at[pl.ds(s * NPT, NPT)])
        pltpu.sync_copy(d2v, d2_hbm.at[pl.ds(s * NPT, NPT)])


WROWS = 80
NWIN = HALF // WROWS
WCELLS = WROWS * NP
WTILE = WCELLS // NS
WPAD = 1024
CCAP = 2080
CROWS = CCAP // ROW_W
EPT = E // NS
SROWS = 50
NCHUNK = TROWS // SROWS


@functools.partial(
    pl.kernel,
    out_type=jax.ShapeDtypeStruct((CELLS,), jnp.float32),
    mesh=_mesh,
    scratch_types=[
        pltpu.VMEM((EPT,), jnp.int32),
        pltpu.VMEM((EPT,), jnp.float32),
        pltpu.VMEM((SROWS, ROW_W), jnp.int32),
        pltpu.VMEM((SROWS, ROW_W), jnp.int32),
        pltpu.VMEM((NP,), jnp.float32),
        pltpu.VMEM((CCAP,), jnp.int32),
        pltpu.VMEM((CCAP,), jnp.float32),
        pltpu.VMEM((CROWS, ROW_W), jnp.int32),
        pltpu.VMEM_SHARED((WCELLS + WPAD,), jnp.float32),
        pltpu.SemaphoreType.DMA,
    ],
    compiler_params=_sc_params,
)
def _abuild_kernel(src_hbm, dst_hbm, dinv_hbm, a_hbm, keyb, normb, ssrc, sdst,
                   dzv, ckey, cnorm, ck2, win, sem):
    c = lax.axis_index("c")
    s = lax.axis_index("s")
    wid = c * NS + s
    lane = lax.iota(jnp.int32, 16)

    pltpu.sync_copy(dinv_hbm, dzv)
    for ch in range(NCHUNK):
        pltpu.sync_copy(src_hbm.at[s, pl.ds(ch * SROWS, SROWS), :], ssrc)
        pltpu.sync_copy(dst_hbm.at[s, pl.ds(ch * SROWS, SROWS), :], sdst)

        def compute(j, carry):
            for l in range(ROW_W // 16):
                sv = ssrc[j, pl.ds(l * 16, 16)]
                dv = sdst[j, pl.ds(l * 16, 16)]
                nm = (plsc.load_gather(dzv, [sv])
                      * plsc.load_gather(dzv, [dv]))
                off = (ch * SROWS + j) * ROW_W + l * 16
                keyb[pl.ds(off, 16)] = dv * NP + sv
                normb[pl.ds(off, 16)] = nm
            return carry
        lax.fori_loop(0, SROWS, compute, 0)

    lo_key = c * HALF * NP
    hi_key = lo_key + HALF * NP

    def press_half(i, cnt):
        kv = keyb[pl.ds(i * 16, 16)]
        nv = normb[pl.ds(i * 16, 16)]
        m = (kv >= lo_key) & (kv < hi_key)
        plsc.store_compressed(keyb.at[pl.ds(cnt, 16)], kv, mask=m)
        plsc.store_compressed(normb.at[pl.ds(cnt, 16)], nv, mask=m)
        return cnt + jnp.max(plsc.all_reduce_population_count(m))
    cnt_h = lax.fori_loop(0, EPT // 16, press_half, jnp.int32(0))
    keyb[pl.ds(cnt_h, 16)] = jnp.full((16,), -1, jnp.int32)
    rows_h = lax.div(cnt_h + 15, 16)

    def zfill(i, carry):
        dzv[pl.ds(i * 16, 16)] = jnp.zeros((16,), jnp.float32)
        return carry
    lax.fori_loop(0, NP // 16, zfill, 0)

    pad0 = WCELLS + wid * 32 + lane

    def window(w, carry):
        keylo = (c * HALF + w * WROWS) * NP

        def zero(z, carry2):
            pltpu.async_copy(dzv, win.at[pl.ds(s * WTILE + z * NP, NP)], sem)
            return carry2
        lax.fori_loop(0, WTILE // NP, zero, 0)

        def zdrain(z, carry2):
            pltpu.make_async_copy(dzv, win.at[pl.ds(s * WTILE, NP)],
                                  sem).wait()
            return carry2
        lax.fori_loop(0, WTILE // NP, zdrain, 0)
        plsc.subcore_barrier()

        def pfill(i, carry2):
            ckey[pl.ds(i * 16, 16)] = pad0
            return carry2
        lax.fori_loop(0, CCAP // 16, pfill, 0)

        def press(i, cnt):
            kv = keyb[pl.ds(i * 16, 16)]
            nv = normb[pl.ds(i * 16, 16)]
            m = (kv >= keylo) & (kv < keylo + WCELLS)
            plsc.store_compressed(ckey.at[pl.ds(cnt, 16)], kv - keylo, mask=m)
            plsc.store_compressed(cnorm.at[pl.ds(cnt, 16)], nv, mask=m)
            pc = jnp.max(plsc.all_reduce_population_count(m))
            return jnp.minimum(cnt + pc, CCAP - 16)
        cnt = lax.fori_loop(0, rows_h, press, jnp.int32(0))
        rows_c = lax.div(cnt + (ROW_W - 1), ROW_W)

        def c2d(r, carry2):
            for l in range(ROW_W // 16):
                ck2[r, pl.ds(l * 16, 16)] = ckey[pl.ds(r * ROW_W + l * 16, 16)]
            return carry2
        lax.fori_loop(0, rows_c, c2d, 0)

        def fire(r, carry2):
            pltpu.async_copy(cnorm.at[pl.ds(r * ROW_W, ROW_W)],
                             win.at[ck2.at[r]], sem, add=True)
            return carry2
        lax.fori_loop(0, rows_c, fire, 0)

        def drain(r, carry2):
            pltpu.make_async_copy(cnorm.at[pl.ds(0, ROW_W)],
                                  win.at[ck2.at[0]], sem).wait()
            return carry2
        lax.fori_loop(0, rows_c, drain, 0)
        plsc.subcore_barrier()

        pltpu.sync_copy(win.at[pl.ds(s * WTILE, WTILE)],
                        a_hbm.at[pl.ds(keylo + s * WTILE, WTILE)])
        return carry
    lax.fori_loop(0, NWIN, window, 0)


def _mm_small(x, w):
    fin, fout = w.shape
    bm = 1024

    def body(x_ref, w_ref, o_ref):
        o_ref[...] = jnp.dot(x_ref[...], w_ref[...],
                             precision=lax.Precision.HIGHEST,
                             preferred_element_type=jnp.float32)

    return pl.pallas_call(
        body,
        grid=(NP // bm,),
        in_specs=[pl.BlockSpec((bm, fin), lambda i: (i, 0)),
                  pl.BlockSpec((fin, fout), lambda i: (0, 0))],
        out_specs=pl.BlockSpec((bm, fout), lambda i: (i, 0)),
        out_shape=jax.ShapeDtypeStruct((NP, fout), jnp.float32),
    )(x, w)


def _gcn_dense(a2, tmp, d2col, brow, relu):
    f = tmp.shape[1]
    bm, bk = 1024, 2048
    kt = NP // bk

    def body(a_ref, t_ref, tm_ref, d2_ref, b_ref, o_ref):
        j = pl.program_id(1)

        @pl.when(j == 0)
        def _():
            o_ref[...] = tm_ref[...] * d2_ref[...] + b_ref[...]

        o_ref[...] += jnp.dot(a_ref[...], t_ref[...],
                              precision=lax.Precision.HIGHEST,
                              preferred_element_type=jnp.float32)
        if relu:
            @pl.when(j == kt - 1)
            def _():
                o_ref[...] = jnp.maximum(o_ref[...], 0.0)

    return pl.pallas_call(
        body,
        grid=(NP // bm, kt),
        in_specs=[pl.BlockSpec((bm, bk), lambda i, j: (i, j)),
                  pl.BlockSpec((bk, f), lambda i, j: (j, 0)),
                  pl.BlockSpec((bm, f), lambda i, j: (i, 0)),
                  pl.BlockSpec((bm, 1), lambda i, j: (i, 0)),
                  pl.BlockSpec((1, f), lambda i, j: (0, 0))],
        out_specs=pl.BlockSpec((bm, f), lambda i, j: (i, 0)),
        out_shape=jax.ShapeDtypeStruct((NP, f), jnp.float32),
        compiler_params=pltpu.CompilerParams(
            dimension_semantics=("parallel", "arbitrary")),
    )(a2, tmp, tmp, d2col, brow)


def kernel(x, edge_index, W1e, b1e, W2e, b2e, W1d, b1d, W2d, b2d):
    src = edge_index[0].reshape(NS, TROWS, ROW_W)
    dst = edge_index[1].reshape(NS, TROWS, ROW_W)

    dinv, d2 = _hist_kernel(dst)
    a_flat = _abuild_kernel(src, dst, dinv)
    a2 = a_flat.reshape(NP, NP)

    d2col = d2.reshape(NP, 1)
    xp = jnp.pad(x, ((0, NP - N), (0, 0)))

    h1 = _gcn_dense(a2, _mm_small(xp, W1e), d2col, b1e.reshape(1, -1), True)
    z = _gcn_dense(a2, _mm_small(h1, W2e), d2col, b2e.reshape(1, -1), False)
    h2 = _gcn_dense(a2, _mm_small(z, W1d), d2col, b1d.reshape(1, -1), True)
    xh = _gcn_dense(a2, _mm_small(h2, W2d), d2col, b2d.reshape(1, -1), False)
    return (xh[:N], z[:N])


# speedup vs baseline: 7.7760x; 1.3365x over previous
import functools

import jax
import jax.numpy as jnp
from jax import lax
from jax.experimental import pallas as pl
from jax.experimental.pallas import tpu as pltpu
from jax.experimental.pallas import tpu_sc as plsc

N = 10000
E = 320000
NP = 10240
NC = 2
NS = 16
CELLS = NP * NP
ROW_W = 80
EROWS = E // ROW_W
TROWS = EROWS // NS
ZB = 20480
NPT = NP // NS
HALF = NP // 2

_mesh = plsc.VectorSubcoreMesh(core_axis_name="c", subcore_axis_name="s")
_sc_params = pltpu.CompilerParams(use_tc_tiling_on_sc=False,
                                  needs_layout_passes=False)


def _rsqrt_vec(x):
    bits = lax.bitcast_convert_type(x, jnp.int32)
    y = lax.bitcast_convert_type(
        jnp.int32(0x5F3759DF) - lax.shift_right_logical(bits, 1), jnp.float32)
    for _ in range(3):
        y = y * (1.5 - 0.5 * x * y * y)
    return y


@functools.partial(
    pl.kernel,
    out_type=(jax.ShapeDtypeStruct((NP,), jnp.float32),
              jax.ShapeDtypeStruct((NP,), jnp.float32)),
    mesh=_mesh,
    scratch_types=[
        pltpu.VMEM((TROWS, ROW_W), jnp.int32),
        pltpu.VMEM((ROW_W,), jnp.float32),
        pltpu.VMEM((NPT,), jnp.float32),
        pltpu.VMEM((NPT,), jnp.float32),
        pltpu.VMEM((NPT,), jnp.float32),
        pltpu.VMEM_SHARED((NP,), jnp.float32),
        pltpu.SemaphoreType.DMA,
    ],
    compiler_params=_sc_params,
)
def _hist_kernel(dst_hbm, dinv_hbm, d2_hbm, dstb, ones_v, hv, dv, d2v, hist,
                 sem):
    c = lax.axis_index("c")
    s = lax.axis_index("s")

    @pl.when(c == 0)
    def _():
        nrows = TROWS
        pltpu.sync_copy(dst_hbm.at[s], dstb)
        for l in range(ROW_W // 16):
            ones_v[pl.ds(l * 16, 16)] = jnp.full((16,), 1.0, jnp.float32)

        def zslice(i, carry):
            hv[pl.ds(i * 16, 16)] = jnp.zeros((16,), jnp.float32)
            return carry
        lax.fori_loop(0, NPT // 16, zslice, 0)
        pltpu.sync_copy(hv, hist.at[pl.ds(s * NPT, NPT)])
        plsc.subcore_barrier()

        def fire(j, carry):
            pltpu.async_copy(ones_v, hist.at[dstb.at[j]], sem, add=True)
            return carry
        lax.fori_loop(0, nrows, fire, 0)

        def drain(j, carry):
            pltpu.make_async_copy(ones_v, hist.at[dstb.at[0]], sem).wait()
            return carry
        lax.fori_loop(0, nrows, drain, 0)
        plsc.subcore_barrier()

        pltpu.sync_copy(hist.at[pl.ds(s * NPT, NPT)], hv)

        def body(i, carry):
            deg = hv[pl.ds(i * 16, 16)] + 1.0
            r = _rsqrt_vec(deg)
            dv[pl.ds(i * 16, 16)] = r
            d2v[pl.ds(i * 16, 16)] = r * r
            return carry
        lax.fori_loop(0, NPT // 16, body, 0)
        pltpu.sync_copy(dv, dinv_hbm.at[pl.ds(s * NPT, NPT)])
        pltpu.sync_copy(d2v, d2_hbm.at[pl.ds(s * NPT, NPT)])


WROWS = 80
NWIN = HALF // WROWS
WCELLS = WROWS * NP
WTILE = WCELLS // NS
WPAD = 1024
CCAP = 2080
CROWS = CCAP // ROW_W
EPT = E // NS
SROWS = 50
NCHUNK = TROWS // SROWS


@functools.partial(
    pl.kernel,
    out_type=jax.ShapeDtypeStruct((CELLS,), jnp.float32),
    mesh=_mesh,
    scratch_types=[
        pltpu.VMEM((EPT,), jnp.int32),
        pltpu.VMEM((EPT,), jnp.float32),
        pltpu.VMEM((SROWS, ROW_W), jnp.int32),
        pltpu.VMEM((SROWS, ROW_W), jnp.int32),
        pltpu.VMEM((NP,), jnp.float32),
        pltpu.VMEM((CCAP,), jnp.int32),
        pltpu.VMEM((CCAP,), jnp.float32),
        pltpu.VMEM((CROWS, ROW_W), jnp.int32),
        pltpu.VMEM_SHARED((WCELLS + WPAD,), jnp.float32),
        pltpu.SemaphoreType.DMA,
    ],
    compiler_params=_sc_params,
)
def _abuild_kernel(src_hbm, dst_hbm, dinv_hbm, a_hbm, keyb, normb, ssrc, sdst,
                   dzv, ckey, cnorm, ck2, win, sem):
    c = lax.axis_index("c")
    s = lax.axis_index("s")
    wid = c * NS + s
    lane = lax.iota(jnp.int32, 16)

    pltpu.sync_copy(dinv_hbm, dzv)
    for ch in range(NCHUNK):
        pltpu.sync_copy(src_hbm.at[s, pl.ds(ch * SROWS, SROWS), :], ssrc)
        pltpu.sync_copy(dst_hbm.at[s, pl.ds(ch * SROWS, SROWS), :], sdst)

        def compute(j, carry):
            for l in range(ROW_W // 16):
                sv = ssrc[j, pl.ds(l * 16, 16)]
                dv = sdst[j, pl.ds(l * 16, 16)]
                nm = (plsc.load_gather(dzv, [sv])
                      * plsc.load_gather(dzv, [dv]))
                off = (ch * SROWS + j) * ROW_W + l * 16
                keyb[pl.ds(off, 16)] = dv * NP + sv
                normb[pl.ds(off, 16)] = nm
            return carry
        lax.fori_loop(0, SROWS, compute, 0)

    lo_key = c * HALF * NP
    hi_key = lo_key + HALF * NP

    def press_half(i, cnt):
        kv = keyb[pl.ds(i * 16, 16)]
        nv = normb[pl.ds(i * 16, 16)]
        m = (kv >= lo_key) & (kv < hi_key)
        plsc.store_compressed(keyb.at[pl.ds(cnt, 16)], kv, mask=m)
        plsc.store_compressed(normb.at[pl.ds(cnt, 16)], nv, mask=m)
        return cnt + jnp.max(plsc.all_reduce_population_count(m))
    cnt_h = lax.fori_loop(0, EPT // 16, press_half, jnp.int32(0))
    keyb[pl.ds(cnt_h, 16)] = jnp.full((16,), -1, jnp.int32)
    rows_h = lax.div(cnt_h + 15, 16)

    def zfill(i, carry):
        dzv[pl.ds(i * 16, 16)] = jnp.zeros((16,), jnp.float32)
        return carry
    lax.fori_loop(0, NP // 16, zfill, 0)

    pad0 = WCELLS + wid * 32 + lane

    def window(w, carry):
        keylo = (c * HALF + w * WROWS) * NP

        def zero(z, carry2):
            pltpu.async_copy(dzv, win.at[pl.ds(s * WTILE + z * NP, NP)], sem)
            return carry2
        lax.fori_loop(0, WTILE // NP, zero, 0)

        def zdrain(z, carry2):
            pltpu.make_async_copy(dzv, win.at[pl.ds(s * WTILE, NP)],
                                  sem).wait()
            return carry2
        lax.fori_loop(0, WTILE // NP, zdrain, 0)
        plsc.subcore_barrier()

        def pfill(i, carry2):
            ckey[pl.ds(i * 16, 16)] = pad0
            return carry2
        lax.fori_loop(0, CCAP // 16, pfill, 0)

        def press(i, cnt):
            kv = keyb[pl.ds(i * 16, 16)]
            nv = normb[pl.ds(i * 16, 16)]
            m = (kv >= keylo) & (kv < keylo + WCELLS)
            plsc.store_compressed(ckey.at[pl.ds(cnt, 16)], kv - keylo, mask=m)
            plsc.store_compressed(cnorm.at[pl.ds(cnt, 16)], nv, mask=m)
            pc = jnp.max(plsc.all_reduce_population_count(m))
            return jnp.minimum(cnt + pc, CCAP - 16)
        cnt = lax.fori_loop(0, rows_h, press, jnp.int32(0))
        rows_c = lax.div(cnt + (ROW_W - 1), ROW_W)

        def c2d(r, carry2):
            for l in range(ROW_W // 16):
                ck2[r, pl.ds(l * 16, 16)] = ckey[pl.ds(r * ROW_W + l * 16, 16)]
            return carry2
        lax.fori_loop(0, rows_c, c2d, 0)

        def fire(r, carry2):
            pltpu.async_copy(cnorm.at[pl.ds(r * ROW_W, ROW_W)],
                             win.at[ck2.at[r]], sem, add=True)
            return carry2
        lax.fori_loop(0, rows_c, fire, 0)

        def drain(r, carry2):
            pltpu.make_async_copy(cnorm.at[pl.ds(0, ROW_W)],
                                  win.at[ck2.at[0]], sem).wait()
            return carry2
        lax.fori_loop(0, rows_c, drain, 0)
        plsc.subcore_barrier()

        pltpu.sync_copy(win.at[pl.ds(s * WTILE, WTILE)],
                        a_hbm.at[pl.ds(keylo + s * WTILE, WTILE)])
        return carry
    lax.fori_loop(0, NWIN, window, 0)


def _mm_small(x, w):
    fin, fout = w.shape
    bm = 1024

    def body(x_ref, w_ref, o_ref):
        o_ref[...] = jnp.dot(x_ref[...], w_ref[...],
                             precision=lax.Precision.HIGHEST,
                             preferred_element_type=jnp.float32)

    return pl.pallas_call(
        body,
        grid=(NP // bm,),
        in_specs=[pl.BlockSpec((bm, fin), lambda i: (i, 0)),
                  pl.BlockSpec((fin, fout), lambda i: (0, 0))],
        out_specs=pl.BlockSpec((bm, fout), lambda i: (i, 0)),
        out_shape=jax.ShapeDtypeStruct((NP, fout), jnp.float32),
    )(x, w)


def _gcn_first(a2, tmp, d2col, brow, relu):
    f = tmp.shape[1]
    bm, bk = 1024, 2048
    kt = NP // bk

    def body(a_ref, t_ref, tm_ref, d2_ref, b_ref, o_ref, abf_ref):
        j = pl.program_id(1)

        @pl.when(j == 0)
        def _():
            o_ref[...] = tm_ref[...] * d2_ref[...] + b_ref[...]

        abf_ref[...] = a_ref[...].astype(jnp.bfloat16)
        o_ref[...] += jnp.dot(a_ref[...], t_ref[...],
                              precision=lax.Precision.HIGHEST,
                              preferred_element_type=jnp.float32)
        if relu:
            @pl.when(j == kt - 1)
            def _():
                o_ref[...] = jnp.maximum(o_ref[...], 0.0)

    return pl.pallas_call(
        body,
        grid=(NP // bm, kt),
        in_specs=[pl.BlockSpec((bm, bk), lambda i, j: (i, j)),
                  pl.BlockSpec((bk, f), lambda i, j: (j, 0)),
                  pl.BlockSpec((bm, f), lambda i, j: (i, 0)),
                  pl.BlockSpec((bm, 1), lambda i, j: (i, 0)),
                  pl.BlockSpec((1, f), lambda i, j: (0, 0))],
        out_specs=[pl.BlockSpec((bm, f), lambda i, j: (i, 0)),
                   pl.BlockSpec((bm, bk), lambda i, j: (i, j))],
        out_shape=[jax.ShapeDtypeStruct((NP, f), jnp.float32),
                   jax.ShapeDtypeStruct((NP, NP), jnp.bfloat16)],
        compiler_params=pltpu.CompilerParams(
            dimension_semantics=("parallel", "arbitrary")),
    )(a2, tmp, tmp, d2col, brow)


def _gcn_dense(a2bf, tmp, d2col, brow, relu):
    f = tmp.shape[1]
    bm, bk = 1024, 2048
    kt = NP // bk

    def body(a_ref, t_ref, tm_ref, d2_ref, b_ref, o_ref):
        j = pl.program_id(1)

        @pl.when(j == 0)
        def _():
            o_ref[...] = tm_ref[...] * d2_ref[...] + b_ref[...]

        o_ref[...] += jnp.dot(a_ref[...], t_ref[...].astype(jnp.bfloat16),
                              preferred_element_type=jnp.float32)
        if relu:
            @pl.when(j == kt - 1)
            def _():
                o_ref[...] = jnp.maximum(o_ref[...], 0.0)

    return pl.pallas_call(
        body,
        grid=(NP // bm, kt),
        in_specs=[pl.BlockSpec((bm, bk), lambda i, j: (i, j)),
                  pl.BlockSpec((bk, f), lambda i, j: (j, 0)),
                  pl.BlockSpec((bm, f), lambda i, j: (i, 0)),
                  pl.BlockSpec((bm, 1), lambda i, j: (i, 0)),
                  pl.BlockSpec((1, f), lambda i, j: (0, 0))],
        out_specs=pl.BlockSpec((bm, f), lambda i, j: (i, 0)),
        out_shape=jax.ShapeDtypeStruct((NP, f), jnp.float32),
        compiler_params=pltpu.CompilerParams(
            dimension_semantics=("parallel", "arbitrary")),
    )(a2bf, tmp, tmp, d2col, brow)


def kernel(x, edge_index, W1e, b1e, W2e, b2e, W1d, b1d, W2d, b2d):
    src = edge_index[0].reshape(NS, TROWS, ROW_W)
    dst = edge_index[1].reshape(NS, TROWS, ROW_W)

    dinv, d2 = _hist_kernel(dst)
    a_flat = _abuild_kernel(src, dst, dinv)
    a2 = a_flat.reshape(NP, NP)

    d2col = d2.reshape(NP, 1)
    xp = jnp.pad(x, ((0, NP - N), (0, 0)))

    h1, a2bf = _gcn_first(a2, _mm_small(xp, W1e), d2col,
                          b1e.reshape(1, -1), True)
    z = _gcn_dense(a2bf, _mm_small(h1, W2e), d2col, b2e.reshape(1, -1), False)
    h2 = _gcn_dense(a2bf, _mm_small(z, W1d), d2col, b1d.reshape(1, -1), True)
    xh = _gcn_dense(a2bf, _mm_small(h2, W2d), d2col, b2d.reshape(1, -1), False)
    return (xh[:N], z[:N])


# speedup vs baseline: 9.3490x; 1.2023x over previous
import functools

import jax
import jax.numpy as jnp
from jax import lax
from jax.experimental import pallas as pl
from jax.experimental.pallas import tpu as pltpu
from jax.experimental.pallas import tpu_sc as plsc

N = 10000
E = 320000
NP = 10240
NC = 2
NS = 16
CELLS = NP * NP
ROW_W = 80
EROWS = E // ROW_W
TROWS = EROWS // NS
ZB = 20480
NPT = NP // NS
HALF = NP // 2

_mesh = plsc.VectorSubcoreMesh(core_axis_name="c", subcore_axis_name="s")
_sc_params = pltpu.CompilerParams(use_tc_tiling_on_sc=False,
                                  needs_layout_passes=False)


def _rsqrt_vec(x):
    bits = lax.bitcast_convert_type(x, jnp.int32)
    y = lax.bitcast_convert_type(
        jnp.int32(0x5F3759DF) - lax.shift_right_logical(bits, 1), jnp.float32)
    for _ in range(3):
        y = y * (1.5 - 0.5 * x * y * y)
    return y


@functools.partial(
    pl.kernel,
    out_type=(jax.ShapeDtypeStruct((NP,), jnp.float32),
              jax.ShapeDtypeStruct((NP,), jnp.float32)),
    mesh=_mesh,
    scratch_types=[
        pltpu.VMEM((TROWS, ROW_W), jnp.int32),
        pltpu.VMEM((ROW_W,), jnp.float32),
        pltpu.VMEM((NPT,), jnp.float32),
        pltpu.VMEM((NPT,), jnp.float32),
        pltpu.VMEM((NPT,), jnp.float32),
        pltpu.VMEM_SHARED((NP,), jnp.float32),
        pltpu.SemaphoreType.DMA,
    ],
    compiler_params=_sc_params,
)
def _hist_kernel(dst_hbm, dinv_hbm, d2_hbm, dstb, ones_v, hv, dv, d2v, hist,
                 sem):
    c = lax.axis_index("c")
    s = lax.axis_index("s")

    @pl.when(c == 0)
    def _():
        nrows = TROWS
        pltpu.sync_copy(dst_hbm.at[s], dstb)
        for l in range(ROW_W // 16):
            ones_v[pl.ds(l * 16, 16)] = jnp.full((16,), 1.0, jnp.float32)

        def zslice(i, carry):
            hv[pl.ds(i * 16, 16)] = jnp.zeros((16,), jnp.float32)
            return carry
        lax.fori_loop(0, NPT // 16, zslice, 0)
        pltpu.sync_copy(hv, hist.at[pl.ds(s * NPT, NPT)])
        plsc.subcore_barrier()

        def fire(j, carry):
            pltpu.async_copy(ones_v, hist.at[dstb.at[j]], sem, add=True)
            return carry
        lax.fori_loop(0, nrows, fire, 0)

        def drain(j, carry):
            pltpu.make_async_copy(ones_v, hist.at[dstb.at[0]], sem).wait()
            return carry
        lax.fori_loop(0, nrows, drain, 0)
        plsc.subcore_barrier()

        pltpu.sync_copy(hist.at[pl.ds(s * NPT, NPT)], hv)

        def body(i, carry):
            deg = hv[pl.ds(i * 16, 16)] + 1.0
            r = _rsqrt_vec(deg)
            dv[pl.ds(i * 16, 16)] = r
            d2v[pl.ds(i * 16, 16)] = r * r
            return carry
        lax.fori_loop(0, NPT // 16, body, 0)
        pltpu.sync_copy(dv, dinv_hbm.at[pl.ds(s * NPT, NPT)])
        pltpu.sync_copy(d2v, d2_hbm.at[pl.ds(s * NPT, NPT)])


WROWS = 64
NWIN = HALF // WROWS
WCELLS = WROWS * NP
WTILE = WCELLS // NS
WPAD = 1024
CCAP = 2080
CROWS = CCAP // ROW_W
EPT = E // NS
SROWS = 50
NCHUNK = TROWS // SROWS
NBUCK = 10
BKEY = 512 * NP
BCAP = 1536


@functools.partial(
    pl.kernel,
    out_type=jax.ShapeDtypeStruct((CELLS,), jnp.float32),
    mesh=_mesh,
    scratch_types=[
        pltpu.VMEM((EPT + 16,), jnp.int32),
        pltpu.VMEM((EPT + 16,), jnp.float32),
        pltpu.VMEM((NBUCK, BCAP), jnp.int32),
        pltpu.VMEM((SROWS, ROW_W), jnp.int32),
        pltpu.VMEM((SROWS, ROW_W), jnp.int32),
        pltpu.VMEM((NP,), jnp.float32),
        pltpu.VMEM((CCAP,), jnp.int32),
        pltpu.VMEM((CCAP,), jnp.float32),
        pltpu.VMEM((CROWS, ROW_W), jnp.int32),
        pltpu.VMEM_SHARED((WCELLS + WPAD,), jnp.float32),
        pltpu.SemaphoreType.DMA,
    ],
    compiler_params=_sc_params,
)
def _abuild_kernel(src_hbm, dst_hbm, dinv_hbm, a_hbm, keyb, normb, sbuck,
                   ssrc, sdst, dzv, ckey, cnorm, ck2, win, sem):
    c = lax.axis_index("c")
    s = lax.axis_index("s")
    wid = c * NS + s
    lane = lax.iota(jnp.int32, 16)

    pltpu.sync_copy(dinv_hbm, dzv)
    for ch in range(NCHUNK):
        pltpu.sync_copy(src_hbm.at[s, pl.ds(ch * SROWS, SROWS), :], ssrc)
        pltpu.sync_copy(dst_hbm.at[s, pl.ds(ch * SROWS, SROWS), :], sdst)

        def compute(j, carry):
            for l in range(ROW_W // 16):
                sv = ssrc[j, pl.ds(l * 16, 16)]
                dv = sdst[j, pl.ds(l * 16, 16)]
                nm = (plsc.load_gather(dzv, [sv])
                      * plsc.load_gather(dzv, [dv]))
                off = (ch * SROWS + j) * ROW_W + l * 16
                keyb[pl.ds(off, 16)] = dv * NP + sv
                normb[pl.ds(off, 16)] = nm
            return carry
        lax.fori_loop(0, SROWS, compute, 0)

    lo_key = c * HALF * NP
    hi_key = lo_key + HALF * NP

    def press_half(i, cnt):
        kv = keyb[pl.ds(i * 16, 16)]
        nv = normb[pl.ds(i * 16, 16)]
        m = (kv >= lo_key) & (kv < hi_key)
        plsc.store_compressed(keyb.at[pl.ds(cnt, 16)], kv, mask=m)
        plsc.store_compressed(normb.at[pl.ds(cnt, 16)], nv, mask=m)
        return cnt + jnp.max(plsc.all_reduce_population_count(m))
    cnt_h = lax.fori_loop(0, EPT // 16, press_half, jnp.int32(0))
    keyb[pl.ds(cnt_h, 16)] = jnp.full((16,), -1, jnp.int32)
    rows_h = lax.div(cnt_h + 15, 16)

    def bbuild(i, cnts):
        kv = keyb[pl.ds(i * 16, 16)]
        slots = lane + i * 16
        out = []
        for b in range(NBUCK):
            m = (kv >= lo_key + b * BKEY) & (kv < lo_key + (b + 1) * BKEY)
            plsc.store_compressed(sbuck.at[b, pl.ds(cnts[b], 16)], slots,
                                  mask=m)
            pc = jnp.max(plsc.all_reduce_population_count(m))
            out.append(jnp.minimum(cnts[b] + pc, BCAP - 16))
        return tuple(out)
    bcnts = lax.fori_loop(0, rows_h, bbuild,
                          tuple(jnp.int32(0) for _ in range(NBUCK)))
    for b in range(NBUCK):
        sbuck[b, pl.ds(bcnts[b], 16)] = jnp.broadcast_to(cnt_h, (16,))

    def zfill(i, carry):
        dzv[pl.ds(i * 16, 16)] = jnp.zeros((16,), jnp.float32)
        return carry
    lax.fori_loop(0, NP // 16, zfill, 0)

    pad0 = WCELLS + wid * 32 + lane

    def window(w, carry):
        keylo = (c * HALF + w * WROWS) * NP
        b_ = lax.shift_right_logical(w, 3)
        bc = jnp.int32(0)
        for b in range(NBUCK):
            bc = jnp.where(b_ == b, bcnts[b], bc)
        rows_b = lax.div(bc + 15, 16)

        def zero(z, carry2):
            pltpu.async_copy(dzv, win.at[pl.ds(s * WTILE + z * NP, NP)], sem)
            return carry2
        lax.fori_loop(0, WTILE // NP, zero, 0)

        def zdrain(z, carry2):
            pltpu.make_async_copy(dzv, win.at[pl.ds(s * WTILE, NP)],
                                  sem).wait()
            return carry2
        lax.fori_loop(0, WTILE // NP, zdrain, 0)
        plsc.subcore_barrier()

        def pfill(i, carry2):
            ckey[pl.ds(i * 16, 16)] = pad0
            return carry2
        lax.fori_loop(0, CCAP // 16, pfill, 0)

        def press(i, cnt):
            slots = sbuck[b_, pl.ds(i * 16, 16)]
            kv = plsc.load_gather(keyb, [slots])
            nv = plsc.load_gather(normb, [slots])
            m = (kv >= keylo) & (kv < keylo + WCELLS)
            plsc.store_compressed(ckey.at[pl.ds(cnt, 16)], kv - keylo, mask=m)
            plsc.store_compressed(cnorm.at[pl.ds(cnt, 16)], nv, mask=m)
            pc = jnp.max(plsc.all_reduce_population_count(m))
            return jnp.minimum(cnt + pc, CCAP - 16)
        cnt = lax.fori_loop(0, rows_b, press, jnp.int32(0))
        rows_c = lax.div(cnt + (ROW_W - 1), ROW_W)

        def c2d(r, carry2):
            for l in range(ROW_W // 16):
                ck2[r, pl.ds(l * 16, 16)] = ckey[pl.ds(r * ROW_W + l * 16, 16)]
            return carry2
        lax.fori_loop(0, rows_c, c2d, 0)

        def fire(r, carry2):
            pltpu.async_copy(cnorm.at[pl.ds(r * ROW_W, ROW_W)],
                             win.at[ck2.at[r]], sem, add=True)
            return carry2
        lax.fori_loop(0, rows_c, fire, 0)

        def drain(r, carry2):
            pltpu.make_async_copy(cnorm.at[pl.ds(0, ROW_W)],
                                  win.at[ck2.at[0]], sem).wait()
            return carry2
        lax.fori_loop(0, rows_c, drain, 0)
        plsc.subcore_barrier()

        pltpu.sync_copy(win.at[pl.ds(s * WTILE, WTILE)],
                        a_hbm.at[pl.ds(keylo + s * WTILE, WTILE)])
        return carry
    lax.fori_loop(0, NWIN, window, 0)


def _mm_small(x, w):
    fin, fout = w.shape
    bm = 1024

    def body(x_ref, w_ref, o_ref):
        o_ref[...] = jnp.dot(x_ref[...], w_ref[...],
                             precision=lax.Precision.HIGHEST,
                             preferred_element_type=jnp.float32)

    return pl.pallas_call(
        body,
        grid=(NP // bm,),
        in_specs=[pl.BlockSpec((bm, fin), lambda i: (i, 0)),
                  pl.BlockSpec((fin, fout), lambda i: (0, 0))],
        out_specs=pl.BlockSpec((bm, fout), lambda i: (i, 0)),
        out_shape=jax.ShapeDtypeStruct((NP, fout), jnp.float32),
    )(x, w)


def _gcn_first(a2, tmp, d2col, brow, relu):
    f = tmp.shape[1]
    bm, bk = 1024, 2048
    kt = NP // bk

    def body(a_ref, t_ref, tm_ref, d2_ref, b_ref, o_ref, abf_ref):
        j = pl.program_id(1)

        @pl.when(j == 0)
        def _():
            o_ref[...] = tm_ref[...] * d2_ref[...] + b_ref[...]

        abf_ref[...] = a_ref[...].astype(jnp.bfloat16)
        o_ref[...] += jnp.dot(a_ref[...], t_ref[...],
                              precision=lax.Precision.HIGHEST,
                              preferred_element_type=jnp.float32)
        if relu:
            @pl.when(j == kt - 1)
            def _():
                o_ref[...] = jnp.maximum(o_ref[...], 0.0)

    return pl.pallas_call(
        body,
        grid=(NP // bm, kt),
        in_specs=[pl.BlockSpec((bm, bk), lambda i, j: (i, j)),
                  pl.BlockSpec((bk, f), lambda i, j: (j, 0)),
                  pl.BlockSpec((bm, f), lambda i, j: (i, 0)),
                  pl.BlockSpec((bm, 1), lambda i, j: (i, 0)),
                  pl.BlockSpec((1, f), lambda i, j: (0, 0))],
        out_specs=[pl.BlockSpec((bm, f), lambda i, j: (i, 0)),
                   pl.BlockSpec((bm, bk), lambda i, j: (i, j))],
        out_shape=[jax.ShapeDtypeStruct((NP, f), jnp.float32),
                   jax.ShapeDtypeStruct((NP, NP), jnp.bfloat16)],
        compiler_params=pltpu.CompilerParams(
            dimension_semantics=("parallel", "arbitrary")),
    )(a2, tmp, tmp, d2col, brow)


def _gcn_dense(a2bf, tmp, d2col, brow, relu):
    f = tmp.shape[1]
    bm, bk = 1024, 2048
    kt = NP // bk

    def body(a_ref, t_ref, tm_ref, d2_ref, b_ref, o_ref):
        j = pl.program_id(1)

        @pl.when(j == 0)
        def _():
            o_ref[...] = tm_ref[...] * d2_ref[...] + b_ref[...]

        o_ref[...] += jnp.dot(a_ref[...], t_ref[...].astype(jnp.bfloat16),
                              preferred_element_type=jnp.float32)
        if relu:
            @pl.when(j == kt - 1)
            def _():
                o_ref[...] = jnp.maximum(o_ref[...], 0.0)

    return pl.pallas_call(
        body,
        grid=(NP // bm, kt),
        in_specs=[pl.BlockSpec((bm, bk), lambda i, j: (i, j)),
                  pl.BlockSpec((bk, f), lambda i, j: (j, 0)),
                  pl.BlockSpec((bm, f), lambda i, j: (i, 0)),
                  pl.BlockSpec((bm, 1), lambda i, j: (i, 0)),
                  pl.BlockSpec((1, f), lambda i, j: (0, 0))],
        out_specs=pl.BlockSpec((bm, f), lambda i, j: (i, 0)),
        out_shape=jax.ShapeDtypeStruct((NP, f), jnp.float32),
        compiler_params=pltpu.CompilerParams(
            dimension_semantics=("parallel", "arbitrary")),
    )(a2bf, tmp, tmp, d2col, brow)


def kernel(x, edge_index, W1e, b1e, W2e, b2e, W1d, b1d, W2d, b2d):
    src = edge_index[0].reshape(NS, TROWS, ROW_W)
    dst = edge_index[1].reshape(NS, TROWS, ROW_W)

    dinv, d2 = _hist_kernel(dst)
    a_flat = _abuild_kernel(src, dst, dinv)
    a2 = a_flat.reshape(NP, NP)

    d2col = d2.reshape(NP, 1)
    xp = jnp.pad(x, ((0, NP - N), (0, 0)))

    h1, a2bf = _gcn_first(a2, _mm_small(xp, W1e), d2col,
                          b1e.reshape(1, -1), True)
    z = _gcn_dense(a2bf, _mm_small(h1, W2e), d2col, b2e.reshape(1, -1), False)
    h2 = _gcn_dense(a2bf, _mm_small(z, W1d), d2col, b1d.reshape(1, -1), True)
    xh = _gcn_dense(a2bf, _mm_small(h2, W2d), d2col, b2d.reshape(1, -1), False)
    return (xh[:N], z[:N])


# speedup vs baseline: 10.2162x; 1.0928x over previous
import functools

import jax
import jax.numpy as jnp
from jax import lax
from jax.experimental import pallas as pl
from jax.experimental.pallas import tpu as pltpu
from jax.experimental.pallas import tpu_sc as plsc

N = 10000
E = 320000
NP = 10240
NC = 2
NS = 16
CELLS = NP * NP
ROW_W = 80
EROWS = E // ROW_W
TROWS = EROWS // NS
ZB = 20480
NPT = NP // NS
HALF = NP // 2

_mesh = plsc.VectorSubcoreMesh(core_axis_name="c", subcore_axis_name="s")
_sc_params = pltpu.CompilerParams(use_tc_tiling_on_sc=False,
                                  needs_layout_passes=False)


def _rsqrt_vec(x):
    bits = lax.bitcast_convert_type(x, jnp.int32)
    y = lax.bitcast_convert_type(
        jnp.int32(0x5F3759DF) - lax.shift_right_logical(bits, 1), jnp.float32)
    for _ in range(3):
        y = y * (1.5 - 0.5 * x * y * y)
    return y


@functools.partial(
    pl.kernel,
    out_type=(jax.ShapeDtypeStruct((NP,), jnp.float32),
              jax.ShapeDtypeStruct((NP,), jnp.float32)),
    mesh=_mesh,
    scratch_types=[
        pltpu.VMEM((TROWS, ROW_W), jnp.int32),
        pltpu.VMEM((ROW_W,), jnp.float32),
        pltpu.VMEM((NPT,), jnp.float32),
        pltpu.VMEM((NPT,), jnp.float32),
        pltpu.VMEM((NPT,), jnp.float32),
        pltpu.VMEM_SHARED((NP,), jnp.float32),
        pltpu.SemaphoreType.DMA,
    ],
    compiler_params=_sc_params,
)
def _hist_kernel(dst_hbm, dinv_hbm, d2_hbm, dstb, ones_v, hv, dv, d2v, hist,
                 sem):
    c = lax.axis_index("c")
    s = lax.axis_index("s")

    @pl.when(c == 0)
    def _():
        nrows = TROWS
        pltpu.sync_copy(dst_hbm.at[s], dstb)
        for l in range(ROW_W // 16):
            ones_v[pl.ds(l * 16, 16)] = jnp.full((16,), 1.0, jnp.float32)

        def zslice(i, carry):
            hv[pl.ds(i * 16, 16)] = jnp.zeros((16,), jnp.float32)
            return carry
        lax.fori_loop(0, NPT // 16, zslice, 0)
        pltpu.sync_copy(hv, hist.at[pl.ds(s * NPT, NPT)])
        plsc.subcore_barrier()

        def fire(j, carry):
            pltpu.async_copy(ones_v, hist.at[dstb.at[j]], sem, add=True)
            return carry
        lax.fori_loop(0, nrows, fire, 0)

        def drain(j, carry):
            pltpu.make_async_copy(ones_v, hist.at[dstb.at[0]], sem).wait()
            return carry
        lax.fori_loop(0, nrows, drain, 0)
        plsc.subcore_barrier()

        pltpu.sync_copy(hist.at[pl.ds(s * NPT, NPT)], hv)

        def body(i, carry):
            deg = hv[pl.ds(i * 16, 16)] + 1.0
            r = _rsqrt_vec(deg)
            dv[pl.ds(i * 16, 16)] = r
            d2v[pl.ds(i * 16, 16)] = r * r
            return carry
        lax.fori_loop(0, NPT // 16, body, 0)
        pltpu.sync_copy(dv, dinv_hbm.at[pl.ds(s * NPT, NPT)])
        pltpu.sync_copy(d2v, d2_hbm.at[pl.ds(s * NPT, NPT)])


WROWS = 64
NWIN = HALF // WROWS
WCELLS = WROWS * NP
WTILE = WCELLS // NS
WPAD = 1024
CCAP = 2080
CROWS = CCAP // ROW_W
EPT = E // NS
SROWS = 50
NCHUNK = TROWS // SROWS
NBUCK = 10
BKEY = 512 * NP
BCAP = 1536


@functools.partial(
    pl.kernel,
    out_type=jax.ShapeDtypeStruct((CELLS,), jnp.float32),
    mesh=_mesh,
    scratch_types=[
        pltpu.VMEM((EPT + 16,), jnp.int32),
        pltpu.VMEM((EPT + 16,), jnp.float32),
        pltpu.VMEM((NBUCK, BCAP), jnp.int32),
        pltpu.VMEM((SROWS, ROW_W), jnp.int32),
        pltpu.VMEM((SROWS, ROW_W), jnp.int32),
        pltpu.VMEM((NP,), jnp.float32),
        pltpu.VMEM((CCAP,), jnp.int32),
        pltpu.VMEM((CCAP,), jnp.float32),
        pltpu.VMEM((CROWS, ROW_W), jnp.int32),
        pltpu.VMEM_SHARED((WCELLS + WPAD,), jnp.float32),
        pltpu.SemaphoreType.DMA,
    ],
    compiler_params=_sc_params,
)
def _abuild_kernel(src_hbm, dst_hbm, dinv_hbm, a_hbm, keyb, normb, sbuck,
                   ssrc, sdst, dzv, ckey, cnorm, ck2, win, sem):
    c = lax.axis_index("c")
    s = lax.axis_index("s")
    wid = c * NS + s
    lane = lax.iota(jnp.int32, 16)

    pltpu.sync_copy(dinv_hbm, dzv)
    for ch in range(NCHUNK):
        pltpu.sync_copy(src_hbm.at[s, pl.ds(ch * SROWS, SROWS), :], ssrc)
        pltpu.sync_copy(dst_hbm.at[s, pl.ds(ch * SROWS, SROWS), :], sdst)

        def compute(j, carry):
            for l in range(ROW_W // 16):
                sv = ssrc[j, pl.ds(l * 16, 16)]
                dv = sdst[j, pl.ds(l * 16, 16)]
                nm = (plsc.load_gather(dzv, [sv])
                      * plsc.load_gather(dzv, [dv]))
                off = (ch * SROWS + j) * ROW_W + l * 16
                keyb[pl.ds(off, 16)] = dv * NP + sv
                normb[pl.ds(off, 16)] = nm
            return carry
        lax.fori_loop(0, SROWS, compute, 0)

    lo_key = c * HALF * NP
    hi_key = lo_key + HALF * NP

    def press_half(i, cnt):
        kv = keyb[pl.ds(i * 16, 16)]
        nv = normb[pl.ds(i * 16, 16)]
        m = (kv >= lo_key) & (kv < hi_key)
        plsc.store_compressed(keyb.at[pl.ds(cnt, 16)], kv, mask=m)
        plsc.store_compressed(normb.at[pl.ds(cnt, 16)], nv, mask=m)
        return cnt + jnp.max(plsc.all_reduce_population_count(m))
    cnt_h = lax.fori_loop(0, EPT // 16, press_half, jnp.int32(0))
    keyb[pl.ds(cnt_h, 16)] = jnp.full((16,), -1, jnp.int32)
    rows_h = lax.div(cnt_h + 15, 16)

    def bbuild(i, cnts):
        kv = keyb[pl.ds(i * 16, 16)]
        slots = lane + i * 16
        out = []
        for b in range(NBUCK):
            m = (kv >= lo_key + b * BKEY) & (kv < lo_key + (b + 1) * BKEY)
            plsc.store_compressed(sbuck.at[b, pl.ds(cnts[b], 16)], slots,
                                  mask=m)
            pc = jnp.max(plsc.all_reduce_population_count(m))
            out.append(jnp.minimum(cnts[b] + pc, BCAP - 16))
        return tuple(out)
    bcnts = lax.fori_loop(0, rows_h, bbuild,
                          tuple(jnp.int32(0) for _ in range(NBUCK)))
    for b in range(NBUCK):
        sbuck[b, pl.ds(bcnts[b], 16)] = jnp.broadcast_to(cnt_h, (16,))

    def zfill(i, carry):
        dzv[pl.ds(i * 16, 16)] = jnp.zeros((16,), jnp.float32)
        return carry
    lax.fori_loop(0, NP // 16, zfill, 0)

    pad0 = WCELLS + wid * 32 + lane

    def window(w, carry):
        keylo = (c * HALF + w * WROWS) * NP
        b_ = lax.shift_right_logical(w, 3)
        bc = jnp.int32(0)
        for b in range(NBUCK):
            bc = jnp.where(b_ == b, bcnts[b], bc)
        rows_b = lax.div(bc + 15, 16)

        def zero(z, carry2):
            pltpu.async_copy(dzv, win.at[pl.ds(s * WTILE + z * NP, NP)], sem)
            return carry2
        lax.fori_loop(0, WTILE // NP, zero, 0)

        def zdrain(z, carry2):
            pltpu.make_async_copy(dzv, win.at[pl.ds(s * WTILE, NP)],
                                  sem).wait()
            return carry2
        lax.fori_loop(0, WTILE // NP, zdrain, 0)
        plsc.subcore_barrier()

        def pfill(i, carry2):
            ckey[pl.ds(i * 16, 16)] = pad0
            return carry2
        lax.fori_loop(0, CCAP // 16, pfill, 0)

        def press(i, cnt):
            slots = sbuck[b_, pl.ds(i * 16, 16)]
            kv = plsc.load_gather(keyb, [slots])
            nv = plsc.load_gather(normb, [slots])
            m = (kv >= keylo) & (kv < keylo + WCELLS)
            plsc.store_compressed(ckey.at[pl.ds(cnt, 16)], kv - keylo, mask=m)
            plsc.store_compressed(cnorm.at[pl.ds(cnt, 16)], nv, mask=m)
            pc = jnp.max(plsc.all_reduce_population_count(m))
            return jnp.minimum(cnt + pc, CCAP - 16)
        cnt = lax.fori_loop(0, rows_b, press, jnp.int32(0))
        rows_c = lax.div(cnt + (ROW_W - 1), ROW_W)

        def c2d(r, carry2):
            for l in range(ROW_W // 16):
                ck2[r, pl.ds(l * 16, 16)] = ckey[pl.ds(r * ROW_W + l * 16, 16)]
            return carry2
        lax.fori_loop(0, rows_c, c2d, 0)

        def fire(r, carry2):
            pltpu.async_copy(cnorm.at[pl.ds(r * ROW_W, ROW_W)],
                             win.at[ck2.at[r]], sem, add=True)
            return carry2
        lax.fori_loop(0, rows_c, fire, 0)

        def drain(r, carry2):
            pltpu.make_async_copy(cnorm.at[pl.ds(0, ROW_W)],
                                  win.at[ck2.at[0]], sem).wait()
            return carry2
        lax.fori_loop(0, rows_c, drain, 0)
        plsc.subcore_barrier()

        pltpu.sync_copy(win.at[pl.ds(s * WTILE, WTILE)],
                        a_hbm.at[pl.ds(keylo + s * WTILE, WTILE)])
        return carry
    lax.fori_loop(0, NWIN, window, 0)


def _mm_small(x, w):
    fin, fout = w.shape
    bm = 1024

    def body(x_ref, w_ref, o_ref):
        o_ref[...] = jnp.dot(x_ref[...], w_ref[...],
                             precision=lax.Precision.HIGHEST,
                             preferred_element_type=jnp.float32)

    return pl.pallas_call(
        body,
        grid=(NP // bm,),
        in_specs=[pl.BlockSpec((bm, fin), lambda i: (i, 0)),
                  pl.BlockSpec((fin, fout), lambda i: (0, 0))],
        out_specs=pl.BlockSpec((bm, fout), lambda i: (i, 0)),
        out_shape=jax.ShapeDtypeStruct((NP, fout), jnp.float32),
    )(x, w)


def _gcn_first(a2, tmp, d2col, brow, relu):
    f = tmp.shape[1]
    bm, bk = 1024, 2048
    kt = NP // bk

    def body(a_ref, t_ref, tm_ref, d2_ref, b_ref, o_ref, abf_ref):
        j = pl.program_id(1)

        @pl.when(j == 0)
        def _():
            o_ref[...] = tm_ref[...] * d2_ref[...] + b_ref[...]

        abf = a_ref[...].astype(jnp.bfloat16)
        abf_ref[...] = abf
        o_ref[...] += jnp.dot(abf, t_ref[...].astype(jnp.bfloat16),
                              preferred_element_type=jnp.float32)
        if relu:
            @pl.when(j == kt - 1)
            def _():
                o_ref[...] = jnp.maximum(o_ref[...], 0.0)

    return pl.pallas_call(
        body,
        grid=(NP // bm, kt),
        in_specs=[pl.BlockSpec((bm, bk), lambda i, j: (i, j)),
                  pl.BlockSpec((bk, f), lambda i, j: (j, 0)),
                  pl.BlockSpec((bm, f), lambda i, j: (i, 0)),
                  pl.BlockSpec((bm, 1), lambda i, j: (i, 0)),
                  pl.BlockSpec((1, f), lambda i, j: (0, 0))],
        out_specs=[pl.BlockSpec((bm, f), lambda i, j: (i, 0)),
                   pl.BlockSpec((bm, bk), lambda i, j: (i, j))],
        out_shape=[jax.ShapeDtypeStruct((NP, f), jnp.float32),
                   jax.ShapeDtypeStruct((NP, NP), jnp.bfloat16)],
        compiler_params=pltpu.CompilerParams(
            dimension_semantics=("parallel", "arbitrary")),
    )(a2, tmp, tmp, d2col, brow)


def _gcn_dense(a2bf, tmp, d2col, brow, relu):
    f = tmp.shape[1]
    bm, bk = 1024, 2048
    kt = NP // bk

    def body(a_ref, t_ref, tm_ref, d2_ref, b_ref, o_ref):
        j = pl.program_id(1)

        @pl.when(j == 0)
        def _():
            o_ref[...] = tm_ref[...] * d2_ref[...] + b_ref[...]

        o_ref[...] += jnp.dot(a_ref[...], t_ref[...].astype(jnp.bfloat16),
                              preferred_element_type=jnp.float32)
        if relu:
            @pl.when(j == kt - 1)
            def _():
                o_ref[...] = jnp.maximum(o_ref[...], 0.0)

    return pl.pallas_call(
        body,
        grid=(NP // bm, kt),
        in_specs=[pl.BlockSpec((bm, bk), lambda i, j: (i, j)),
                  pl.BlockSpec((bk, f), lambda i, j: (j, 0)),
                  pl.BlockSpec((bm, f), lambda i, j: (i, 0)),
                  pl.BlockSpec((bm, 1), lambda i, j: (i, 0)),
                  pl.BlockSpec((1, f), lambda i, j: (0, 0))],
        out_specs=pl.BlockSpec((bm, f), lambda i, j: (i, 0)),
        out_shape=jax.ShapeDtypeStruct((NP, f), jnp.float32),
        compiler_params=pltpu.CompilerParams(
            dimension_semantics=("parallel", "arbitrary")),
    )(a2bf, tmp, tmp, d2col, brow)


def kernel(x, edge_index, W1e, b1e, W2e, b2e, W1d, b1d, W2d, b2d):
    src = edge_index[0].reshape(NS, TROWS, ROW_W)
    dst = edge_index[1].reshape(NS, TROWS, ROW_W)

    dinv, d2 = _hist_kernel(dst)
    a_flat = _abuild_kernel(src, dst, dinv)
    a2 = a_flat.reshape(NP, NP)

    d2col = d2.reshape(NP, 1)
    xp = jnp.pad(x, ((0, NP - N), (0, 0)))

    h1, a2bf = _gcn_first(a2, _mm_small(xp, W1e), d2col,
                          b1e.reshape(1, -1), True)
    z = _gcn_dense(a2bf, _mm_small(h1, W2e), d2col, b2e.reshape(1, -1), False)
    h2 = _gcn_dense(a2bf, _mm_small(z, W1d), d2col, b1d.reshape(1, -1), True)
    xh = _gcn_dense(a2bf, _mm_small(h2, W2d), d2col, b2d.reshape(1, -1), False)
    return (xh[:N], z[:N])


# speedup vs baseline: 10.5408x; 1.0318x over previous
import functools

import jax
import jax.numpy as jnp
from jax import lax
from jax.experimental import pallas as pl
from jax.experimental.pallas import tpu as pltpu
from jax.experimental.pallas import tpu_sc as plsc

N = 10000
E = 320000
NP = 10240
NC = 2
NS = 16
CELLS = NP * NP
ROW_W = 80
EROWS = E // ROW_W
TROWS = EROWS // NS
ZB = 20480
NPT = NP // NS
HALF = NP // 2

_mesh = plsc.VectorSubcoreMesh(core_axis_name="c", subcore_axis_name="s")
_sc_params = pltpu.CompilerParams(use_tc_tiling_on_sc=False,
                                  needs_layout_passes=False)


def _rsqrt_vec(x):
    bits = lax.bitcast_convert_type(x, jnp.int32)
    y = lax.bitcast_convert_type(
        jnp.int32(0x5F3759DF) - lax.shift_right_logical(bits, 1), jnp.float32)
    for _ in range(3):
        y = y * (1.5 - 0.5 * x * y * y)
    return y


@functools.partial(
    pl.kernel,
    out_type=(jax.ShapeDtypeStruct((NP,), jnp.float32),
              jax.ShapeDtypeStruct((NP,), jnp.float32)),
    mesh=_mesh,
    scratch_types=[
        pltpu.VMEM((TROWS, ROW_W), jnp.int32),
        pltpu.VMEM((ROW_W,), jnp.float32),
        pltpu.VMEM((NPT,), jnp.float32),
        pltpu.VMEM((NPT,), jnp.float32),
        pltpu.VMEM((NPT,), jnp.float32),
        pltpu.VMEM_SHARED((NP,), jnp.float32),
        pltpu.SemaphoreType.DMA,
    ],
    compiler_params=_sc_params,
)
def _hist_kernel(dst_hbm, dinv_hbm, d2_hbm, dstb, ones_v, hv, dv, d2v, hist,
                 sem):
    c = lax.axis_index("c")
    s = lax.axis_index("s")

    @pl.when(c == 0)
    def _():
        nrows = TROWS
        pltpu.sync_copy(dst_hbm.at[s], dstb)
        for l in range(ROW_W // 16):
            ones_v[pl.ds(l * 16, 16)] = jnp.full((16,), 1.0, jnp.float32)

        def zslice(i, carry):
            hv[pl.ds(i * 16, 16)] = jnp.zeros((16,), jnp.float32)
            return carry
        lax.fori_loop(0, NPT // 16, zslice, 0)
        pltpu.sync_copy(hv, hist.at[pl.ds(s * NPT, NPT)])
        plsc.subcore_barrier()

        def fire(j, carry):
            pltpu.async_copy(ones_v, hist.at[dstb.at[j]], sem, add=True)
            return carry
        lax.fori_loop(0, nrows, fire, 0)

        def drain(j, carry):
            pltpu.make_async_copy(ones_v, hist.at[dstb.at[0]], sem).wait()
            return carry
        lax.fori_loop(0, nrows, drain, 0)
        plsc.subcore_barrier()

        pltpu.sync_copy(hist.at[pl.ds(s * NPT, NPT)], hv)

        def body(i, carry):
            deg = hv[pl.ds(i * 16, 16)] + 1.0
            r = _rsqrt_vec(deg)
            dv[pl.ds(i * 16, 16)] = r
            d2v[pl.ds(i * 16, 16)] = r * r
            return carry
        lax.fori_loop(0, NPT // 16, body, 0)
        pltpu.sync_copy(dv, dinv_hbm.at[pl.ds(s * NPT, NPT)])
        pltpu.sync_copy(d2v, d2_hbm.at[pl.ds(s * NPT, NPT)])


WROWS = 64
NWIN = HALF // WROWS
WCELLS = WROWS * NP
WTILE = WCELLS // NS
WPAD = 1024
CCAP = 2080
CROWS = CCAP // ROW_W
EPT = E // NS
SROWS = 50
NCHUNK = TROWS // SROWS
NBUCK = 10
BKEY = 512 * NP
BCAP = 1536


@functools.partial(
    pl.kernel,
    out_type=jax.ShapeDtypeStruct((CELLS,), jnp.float32),
    mesh=_mesh,
    scratch_types=[
        pltpu.VMEM((EPT + 16,), jnp.int32),
        pltpu.VMEM((EPT + 16,), jnp.float32),
        pltpu.VMEM((NBUCK, BCAP), jnp.int32),
        pltpu.VMEM((SROWS, ROW_W), jnp.int32),
        pltpu.VMEM((SROWS, ROW_W), jnp.int32),
        pltpu.VMEM((NP,), jnp.float32),
        pltpu.VMEM((CCAP,), jnp.int32),
        pltpu.VMEM((CCAP,), jnp.float32),
        pltpu.VMEM((CROWS, ROW_W), jnp.int32),
        pltpu.VMEM_SHARED((WCELLS + WPAD,), jnp.float32),
        pltpu.SemaphoreType.DMA,
    ],
    compiler_params=_sc_params,
)
def _abuild_kernel(src_hbm, dst_hbm, dinv_hbm, a_hbm, keyb, normb, sbuck,
                   ssrc, sdst, dzv, ckey, cnorm, ck2, win, sem):
    c = lax.axis_index("c")
    s = lax.axis_index("s")
    wid = c * NS + s
    lane = lax.iota(jnp.int32, 16)

    pltpu.sync_copy(dinv_hbm, dzv)
    for ch in range(NCHUNK):
        pltpu.sync_copy(src_hbm.at[s, pl.ds(ch * SROWS, SROWS), :], ssrc)
        pltpu.sync_copy(dst_hbm.at[s, pl.ds(ch * SROWS, SROWS), :], sdst)

        def compute(j, carry):
            for l in range(ROW_W // 16):
                sv = ssrc[j, pl.ds(l * 16, 16)]
                dv = sdst[j, pl.ds(l * 16, 16)]
                nm = (plsc.load_gather(dzv, [sv])
                      * plsc.load_gather(dzv, [dv]))
                off = (ch * SROWS + j) * ROW_W + l * 16
                keyb[pl.ds(off, 16)] = dv * NP + sv
                normb[pl.ds(off, 16)] = nm
            return carry
        lax.fori_loop(0, SROWS, compute, 0)

    lo_key = c * HALF * NP
    hi_key = lo_key + HALF * NP

    def press_half(i, cnt):
        kv = keyb[pl.ds(i * 16, 16)]
        nv = normb[pl.ds(i * 16, 16)]
        m = (kv >= lo_key) & (kv < hi_key)
        plsc.store_compressed(keyb.at[pl.ds(cnt, 16)], kv, mask=m)
        plsc.store_compressed(normb.at[pl.ds(cnt, 16)], nv, mask=m)
        return cnt + jnp.max(plsc.all_reduce_population_count(m))
    cnt_h = lax.fori_loop(0, EPT // 16, press_half, jnp.int32(0))
    keyb[pl.ds(cnt_h, 16)] = jnp.full((16,), -1, jnp.int32)
    rows_h = lax.div(cnt_h + 15, 16)

    def bbuild(i, cnts):
        kv = keyb[pl.ds(i * 16, 16)]
        slots = lane + i * 16
        out = []
        for b in range(NBUCK):
            m = (kv >= lo_key + b * BKEY) & (kv < lo_key + (b + 1) * BKEY)
            plsc.store_compressed(sbuck.at[b, pl.ds(cnts[b], 16)], slots,
                                  mask=m)
            pc = jnp.max(plsc.all_reduce_population_count(m))
            out.append(jnp.minimum(cnts[b] + pc, BCAP - 16))
        return tuple(out)
    bcnts = lax.fori_loop(0, rows_h, bbuild,
                          tuple(jnp.int32(0) for _ in range(NBUCK)))
    for b in range(NBUCK):
        sbuck[b, pl.ds(bcnts[b], 16)] = jnp.broadcast_to(cnt_h, (16,))

    def zfill(i, carry):
        dzv[pl.ds(i * 16, 16)] = jnp.zeros((16,), jnp.float32)
        return carry
    lax.fori_loop(0, NP // 16, zfill, 0)

    pad0 = WCELLS + wid * 32 + lane

    def window(w, carry):
        keylo = (c * HALF + w * WROWS) * NP
        b_ = lax.shift_right_logical(w, 3)
        bc = jnp.int32(0)
        for b in range(NBUCK):
            bc = jnp.where(b_ == b, bcnts[b], bc)
        rows_b = lax.div(bc + 15, 16)

        def zero(z, carry2):
            pltpu.async_copy(dzv, win.at[pl.ds(s * WTILE + z * NP, NP)], sem)
            return carry2
        lax.fori_loop(0, WTILE // NP, zero, 0)

        def zdrain(z, carry2):
            pltpu.make_async_copy(dzv, win.at[pl.ds(s * WTILE, NP)],
                                  sem).wait()
            return carry2
        lax.fori_loop(0, WTILE // NP, zdrain, 0)
        plsc.subcore_barrier()

        def pfill(i, carry2):
            ckey[pl.ds(i * 16, 16)] = pad0
            return carry2
        lax.fori_loop(0, CCAP // 16, pfill, 0)

        def press(i, cnt):
            slots = sbuck[b_, pl.ds(i * 16, 16)]
            kv = plsc.load_gather(keyb, [slots])
            nv = plsc.load_gather(normb, [slots])
            m = (kv >= keylo) & (kv < keylo + WCELLS)
            plsc.store_compressed(ckey.at[pl.ds(cnt, 16)], kv - keylo, mask=m)
            plsc.store_compressed(cnorm.at[pl.ds(cnt, 16)], nv, mask=m)
            pc = jnp.max(plsc.all_reduce_population_count(m))
            return jnp.minimum(cnt + pc, CCAP - 16)
        cnt = lax.fori_loop(0, rows_b, press, jnp.int32(0))
        rows_c = lax.div(cnt + (ROW_W - 1), ROW_W)

        def c2d(r, carry2):
            for l in range(ROW_W // 16):
                ck2[r, pl.ds(l * 16, 16)] = ckey[pl.ds(r * ROW_W + l * 16, 16)]
            return carry2
        lax.fori_loop(0, rows_c, c2d, 0)

        def fire(r, carry2):
            pltpu.async_copy(cnorm.at[pl.ds(r * ROW_W, ROW_W)],
                             win.at[ck2.at[r]], sem, add=True)
            return carry2
        lax.fori_loop(0, rows_c, fire, 0)

        def drain(r, carry2):
            pltpu.make_async_copy(cnorm.at[pl.ds(0, ROW_W)],
                                  win.at[ck2.at[0]], sem).wait()
            return carry2
        lax.fori_loop(0, rows_c, drain, 0)
        plsc.subcore_barrier()

        pltpu.sync_copy(win.at[pl.ds(s * WTILE, WTILE)],
                        a_hbm.at[pl.ds(keylo + s * WTILE, WTILE)])
        return carry
    lax.fori_loop(0, NWIN, window, 0)


def _mm_small(x, w):
    fin, fout = w.shape
    bm = 1024

    def body(x_ref, w_ref, o_ref):
        o_ref[...] = jnp.dot(x_ref[...].astype(jnp.bfloat16),
                             w_ref[...].astype(jnp.bfloat16),
                             preferred_element_type=jnp.float32
                             ).astype(jnp.bfloat16)

    return pl.pallas_call(
        body,
        grid=(NP // bm,),
        in_specs=[pl.BlockSpec((bm, fin), lambda i: (i, 0)),
                  pl.BlockSpec((fin, fout), lambda i: (0, 0))],
        out_specs=pl.BlockSpec((bm, fout), lambda i: (i, 0)),
        out_shape=jax.ShapeDtypeStruct((NP, fout), jnp.bfloat16),
    )(x, w)


def _gcn_first(a2, tmp, d2col, brow, relu):
    f = tmp.shape[1]
    bm, bk = 1024, 2048
    kt = NP // bk

    def body(a_ref, t_ref, tm_ref, d2_ref, b_ref, o_ref, abf_ref):
        j = pl.program_id(1)

        @pl.when(j == 0)
        def _():
            o_ref[...] = (tm_ref[...].astype(jnp.float32) * d2_ref[...]
                          + b_ref[...])

        abf = a_ref[...].astype(jnp.bfloat16)
        abf_ref[...] = abf
        o_ref[...] += jnp.dot(abf, t_ref[...],
                              preferred_element_type=jnp.float32)
        if relu:
            @pl.when(j == kt - 1)
            def _():
                o_ref[...] = jnp.maximum(o_ref[...], 0.0)

    return pl.pallas_call(
        body,
        grid=(NP // bm, kt),
        in_specs=[pl.BlockSpec((bm, bk), lambda i, j: (i, j)),
                  pl.BlockSpec((bk, f), lambda i, j: (j, 0)),
                  pl.BlockSpec((bm, f), lambda i, j: (i, 0)),
                  pl.BlockSpec((bm, 1), lambda i, j: (i, 0)),
                  pl.BlockSpec((1, f), lambda i, j: (0, 0))],
        out_specs=[pl.BlockSpec((bm, f), lambda i, j: (i, 0)),
                   pl.BlockSpec((bm, bk), lambda i, j: (i, j))],
        out_shape=[jax.ShapeDtypeStruct((NP, f), jnp.float32),
                   jax.ShapeDtypeStruct((NP, NP), jnp.bfloat16)],
        compiler_params=pltpu.CompilerParams(
            dimension_semantics=("parallel", "arbitrary")),
    )(a2, tmp, tmp, d2col, brow)


def _gcn_dense(a2bf, tmp, d2col, brow, relu):
    f = tmp.shape[1]
    bm, bk = 1024, 2048
    kt = NP // bk

    def body(a_ref, t_ref, tm_ref, d2_ref, b_ref, o_ref):
        j = pl.program_id(1)

        @pl.when(j == 0)
        def _():
            o_ref[...] = (tm_ref[...].astype(jnp.float32) * d2_ref[...]
                          + b_ref[...])

        o_ref[...] += jnp.dot(a_ref[...], t_ref[...],
                              preferred_element_type=jnp.float32)
        if relu:
            @pl.when(j == kt - 1)
            def _():
                o_ref[...] = jnp.maximum(o_ref[...], 0.0)

    return pl.pallas_call(
        body,
        grid=(NP // bm, kt),
        in_specs=[pl.BlockSpec((bm, bk), lambda i, j: (i, j)),
                  pl.BlockSpec((bk, f), lambda i, j: (j, 0)),
                  pl.BlockSpec((bm, f), lambda i, j: (i, 0)),
                  pl.BlockSpec((bm, 1), lambda i, j: (i, 0)),
                  pl.BlockSpec((1, f), lambda i, j: (0, 0))],
        out_specs=pl.BlockSpec((bm, f), lambda i, j: (i, 0)),
        out_shape=jax.ShapeDtypeStruct((NP, f), jnp.float32),
        compiler_params=pltpu.CompilerParams(
            dimension_semantics=("parallel", "arbitrary")),
    )(a2bf, tmp, tmp, d2col, brow)


def kernel(x, edge_index, W1e, b1e, W2e, b2e, W1d, b1d, W2d, b2d):
    src = edge_index[0].reshape(NS, TROWS, ROW_W)
    dst = edge_index[1].reshape(NS, TROWS, ROW_W)

    dinv, d2 = _hist_kernel(dst)
    a_flat = _abuild_kernel(src, dst, dinv)
    a2 = a_flat.reshape(NP, NP)

    d2col = d2.reshape(NP, 1)
    xp = jnp.pad(x, ((0, NP - N), (0, 0)))

    h1, a2bf = _gcn_first(a2, _mm_small(xp, W1e), d2col,
                          b1e.reshape(1, -1), True)
    z = _gcn_dense(a2bf, _mm_small(h1, W2e), d2col, b2e.reshape(1, -1), False)
    h2 = _gcn_dense(a2bf, _mm_small(z, W1d), d2col, b1d.reshape(1, -1), True)
    xh = _gcn_dense(a2bf, _mm_small(h2, W2d), d2col, b2d.reshape(1, -1), False)
    return (xh[:N], z[:N])
